# Initial kernel scaffold; baseline (speedup 1.0000x reference)
import jax
import jax.numpy as jnp
from jax.experimental import pallas as pl


def kernel(input_graphs, w, gammat):
    raise NotImplementedError("write your pallas kernel here")


# speedup vs baseline: 5.2938x; 5.2938x over previous
import jax
import jax.numpy as jnp
from jax.experimental import pallas as pl
from jax.experimental.pallas import tpu as pltpu

_NUM_NODES = 100
_TIME_STEPS = 64
_NUM_FILTERS = 32
_FILTER_SIZE = 4
_C = _NUM_NODES * _NUM_NODES
_OUT_POS = _TIME_STEPS - _FILTER_SIZE + 1


def _tgcnn_kernel(gam_ref, x_ref, w_ref, o_ref):
    neg_gamma = -gam_ref[0, 0]
    xb = x_ref[0]
    xv = jnp.where(xb != 0.0, jnp.exp(xb * neg_gamma), 0.0)
    acc = jax.lax.dot_general(
        w_ref[...], xv,
        dimension_numbers=(((0,), (0,)), ((), ())),
        preferred_element_type=jnp.float32)
    o_ref[0] = (acc[0:32, 0:61] + acc[32:64, 1:62]
                + acc[64:96, 2:63] + acc[96:128, 3:64])


def kernel(input_graphs, w, gammat):
    b = input_graphs.shape[0]
    x2 = input_graphs.reshape(b, _C, _TIME_STEPS)
    wf = w.reshape(_C, _FILTER_SIZE * _NUM_FILTERS)
    gamma = 10.0 * jax.nn.sigmoid(gammat)

    out = pl.pallas_call(
        _tgcnn_kernel,
        grid=(b,),
        in_specs=[
            pl.BlockSpec((1, 1), lambda i: (0, 0), memory_space=pltpu.SMEM),
            pl.BlockSpec((1, _C, _TIME_STEPS), lambda i: (i, 0, 0)),
            pl.BlockSpec((_C, _FILTER_SIZE * _NUM_FILTERS), lambda i: (0, 0)),
        ],
        out_specs=pl.BlockSpec((1, _NUM_FILTERS, _OUT_POS), lambda i: (i, 0, 0)),
        out_shape=jax.ShapeDtypeStruct((b, _NUM_FILTERS, _OUT_POS), jnp.float32),
    )(gamma, x2, wf)
    return out[:, :, None, :]


# speedup vs baseline: 5.4949x; 1.0380x over previous
import jax
import jax.numpy as jnp
from jax.experimental import pallas as pl
from jax.experimental.pallas import tpu as pltpu

_NUM_NODES = 100
_TIME_STEPS = 64
_NUM_FILTERS = 32
_FILTER_SIZE = 4
_C = _NUM_NODES * _NUM_NODES
_OUT_POS = _TIME_STEPS - _FILTER_SIZE + 1
_NSPLIT = 4
_CSUB = _C // _NSPLIT


def _tgcnn_kernel(gam_ref, *refs):
    x_refs = refs[:_NSPLIT]
    w_ref = refs[_NSPLIT]
    o_ref = refs[_NSPLIT + 1]
    neg_gamma = -gam_ref[0, 0]
    acc = jnp.zeros((_FILTER_SIZE * _NUM_FILTERS, _TIME_STEPS), jnp.float32)
    for q in range(_NSPLIT):
        xb = x_refs[q][0, 0]
        xv = jnp.where(xb != 0.0, jnp.exp(xb * neg_gamma), 0.0)
        acc = acc + jax.lax.dot_general(
            w_ref[q], xv,
            dimension_numbers=(((0,), (0,)), ((), ())),
            preferred_element_type=jnp.float32)
    o_ref[0] = (acc[0:32, 0:61] + acc[32:64, 1:62]
                + acc[64:96, 2:63] + acc[96:128, 3:64])


def kernel(input_graphs, w, gammat):
    b = input_graphs.shape[0]
    x4 = input_graphs.reshape(b, _NSPLIT, _CSUB, _TIME_STEPS)
    wf = w.reshape(_NSPLIT, _CSUB, _FILTER_SIZE * _NUM_FILTERS)
    gamma = 10.0 * jax.nn.sigmoid(gammat)

    x_specs = [
        pl.BlockSpec((1, 1, _CSUB, _TIME_STEPS), lambda i, q=q: (i, q, 0, 0))
        for q in range(_NSPLIT)
    ]
    out = pl.pallas_call(
        _tgcnn_kernel,
        grid=(b,),
        in_specs=[
            pl.BlockSpec((1, 1), lambda i: (0, 0), memory_space=pltpu.SMEM),
            *x_specs,
            pl.BlockSpec((_NSPLIT, _CSUB, _FILTER_SIZE * _NUM_FILTERS),
                         lambda i: (0, 0, 0)),
        ],
        out_specs=pl.BlockSpec((1, _NUM_FILTERS, _OUT_POS), lambda i: (i, 0, 0)),
        out_shape=jax.ShapeDtypeStruct((b, _NUM_FILTERS, _OUT_POS), jnp.float32),
    )(gamma, *([x4] * _NSPLIT), wf)
    return out[:, :, None, :]


# speedup vs baseline: 6.7931x; 1.2362x over previous
import jax
import jax.numpy as jnp
from jax.experimental import pallas as pl
from jax.experimental.pallas import tpu as pltpu

_NUM_NODES = 100
_TIME_STEPS = 64
_NUM_FILTERS = 32
_FILTER_SIZE = 4
_C = _NUM_NODES * _NUM_NODES
_OUT_POS = _TIME_STEPS - _FILTER_SIZE + 1


def _tgcnn_kernel(gam_ref, x_ref, w_ref, o_ref):
    neg_gamma = -gam_ref[0, 0]
    xb = x_ref[0].reshape(_C, _TIME_STEPS)
    xv = jnp.where(xb != 0.0, jnp.exp(xb * neg_gamma), 0.0)
    acc = jax.lax.dot_general(
        w_ref[...], xv,
        dimension_numbers=(((0,), (0,)), ((), ())),
        preferred_element_type=jnp.float32)
    o_ref[0] = (acc[0:32, 0:61] + acc[32:64, 1:62]
                + acc[64:96, 2:63] + acc[96:128, 3:64])


def kernel(input_graphs, w, gammat):
    b = input_graphs.shape[0]
    wf = w.reshape(_C, _FILTER_SIZE * _NUM_FILTERS)
    gamma = 10.0 * jax.nn.sigmoid(gammat)

    out = pl.pallas_call(
        _tgcnn_kernel,
        grid=(b,),
        in_specs=[
            pl.BlockSpec((1, 1), lambda i: (0, 0), memory_space=pltpu.SMEM),
            pl.BlockSpec((1, _NUM_NODES, _NUM_NODES, _TIME_STEPS),
                         lambda i: (i, 0, 0, 0)),
            pl.BlockSpec((_C, _FILTER_SIZE * _NUM_FILTERS), lambda i: (0, 0)),
        ],
        out_specs=pl.BlockSpec((1, _NUM_FILTERS, _OUT_POS), lambda i: (i, 0, 0)),
        out_shape=jax.ShapeDtypeStruct((b, _NUM_FILTERS, _OUT_POS), jnp.float32),
    )(gamma, input_graphs, wf)
    return out[:, :, None, :]
